# baseline (speedup 1.0000x reference)
import functools

import jax
import jax.numpy as jnp
from jax import lax
from jax.experimental import pallas as pl
from jax.experimental.pallas import tpu as pltpu
from jax.experimental.pallas import tpu_sc as plsc

_N = 10000
_E = 160000
_L = 128
_NC = 2
_NS = 16
_K = 128
_NB = 80
_NH = 5
_NBH = _NB // _NH
_EPT = _NB * _K
_EPAD = _NS * _EPT
_PAD_ROWS = 8
_RPT = 640
_RLAST = _N - (_NS - 1) * _RPT


def _split_rows_copy(s, make_src, make_dst):
  row0 = pl.multiple_of(s * _RPT, 8)

  @pl.when(s < _NS - 1)
  def _():
    pltpu.sync_copy(make_src(row0, _RPT), make_dst(row0, _RPT))

  @pl.when(s == _NS - 1)
  def _():
    off = (_NS - 1) * _RPT
    pltpu.sync_copy(make_src(off, _RLAST), make_dst(off, _RLAST))


def _make_segsum(c_chunks: int, with_count: bool):
  n_pass = c_chunks // _NC
  mesh = plsc.VectorSubcoreMesh(
      core_axis_name="c", subcore_axis_name="s",
      num_cores=_NC, num_subcores=_NS)

  out_type = [jax.ShapeDtypeStruct((c_chunks, _N, _L), jnp.float32)]
  scratch = [
      pltpu.VMEM((_NB, _K), jnp.int32),
      pltpu.VMEM((_NBH, _K), jnp.int32),
      pltpu.VMEM((2, _K, _L), jnp.float32),
      pltpu.VMEM_SHARED((_N + _PAD_ROWS, _L), jnp.float32),
      pltpu.SemaphoreType.DMA,
      pltpu.SemaphoreType.DMA,
      pltpu.SemaphoreType.DMA,
      pltpu.SemaphoreType.DMA,
  ]
  if with_count:
    out_type.append(jax.ShapeDtypeStruct((_NC, _N, _L), jnp.float32))

  def body(*refs):
    if with_count:
      (table, srcp, dstp, zeros, ones,
       out, cnt,
       src_v, dst_v, rows_v, acc, sem0, sem1, sem2, sem3) = refs
    else:
      (table, srcp, dstp, zeros,
       out,
       src_v, dst_v, rows_v, acc, sem0, sem1, sem2, sem3) = refs
    c = lax.axis_index("c")
    s = lax.axis_index("s")

    pltpu.sync_copy(srcp.at[s], src_v)

    for p in range(n_pass):
      chunk = p * _NC + c
      _split_rows_copy(s, lambda o, n: zeros.at[pl.ds(o, n)],
                       lambda o, n: acc.at[pl.ds(o, n)])
      plsc.subcore_barrier()

      def gather(g, buf, sem):
        return pltpu.make_async_copy(
            table.at[chunk].at[src_v.at[g]], rows_v.at[buf], sem)

      def scatter(b, buf, sem):
        return pltpu.make_async_copy(rows_v.at[buf], acc.at[dst_v.at[b]], sem)

      gsem = (sem0, sem1)
      ssem = (sem2, sem3)
      gather(0, 0, sem0).start()
      for half in range(_NH):
        if half > 0:
          scatter(_NBH - 1, (_NBH - 1) % 2, ssem[(_NBH - 1) % 2]).wait()
        pltpu.sync_copy(dstp.at[s, pl.ds(half * _NBH, _NBH)], dst_v)

        def batch_body(b, carry):
          g = half * _NBH + b

          def step(buf):
            other = 1 - buf
            gather(g, buf, gsem[buf]).wait()
            scatter(b, buf, ssem[buf]).start(add=True)

            @pl.when(g + 1 < _NB)
            def _():
              @pl.when(b >= 1)
              def _():
                scatter(b - 1, other, ssem[other]).wait()
              gather(g + 1, other, gsem[other]).start()

          @pl.when(b % 2 == 0)
          def _():
            step(0)

          @pl.when(b % 2 == 1)
          def _():
            step(1)
          return carry
        lax.fori_loop(0, _NBH, batch_body, 0)

      scatter(_NBH - 2, (_NBH - 2) % 2, ssem[(_NBH - 2) % 2]).wait()
      scatter(_NBH - 1, (_NBH - 1) % 2, ssem[(_NBH - 1) % 2]).wait()
      plsc.subcore_barrier()
      _split_rows_copy(s, lambda o, n: acc.at[pl.ds(o, n)],
                       lambda o, n: out.at[chunk, pl.ds(o, n)])
      if p != n_pass - 1:
        plsc.subcore_barrier()

    if with_count:
      plsc.subcore_barrier()
      _split_rows_copy(s, lambda o, n: zeros.at[pl.ds(o, n)],
                       lambda o, n: acc.at[pl.ds(o, n)])
      pltpu.sync_copy(ones, rows_v.at[0])
      plsc.subcore_barrier()
      for half in range(_NH):
        @pl.when(c == half % _NC)
        def _():
          pltpu.sync_copy(dstp.at[s, pl.ds(half * _NBH, _NBH)], dst_v)

          def cnt_start(b, carry):
            pltpu.make_async_copy(
                rows_v.at[0], acc.at[dst_v.at[b]], sem0).start(add=True)
            return carry
          lax.fori_loop(0, _NBH, cnt_start, 0)

          def cnt_drain(b, carry):
            pltpu.make_async_copy(
                rows_v.at[0], acc.at[dst_v.at[b]], sem0).wait()
            return carry
          lax.fori_loop(0, _NBH, cnt_drain, 0)
      plsc.subcore_barrier()
      _split_rows_copy(s, lambda o, n: acc.at[pl.ds(o, n)],
                       lambda o, n: cnt.at[c, pl.ds(o, n)])

  return pl.kernel(body, out_type=out_type, mesh=mesh, scratch_types=scratch)


_segsum2 = _make_segsum(2, with_count=True)
_segsum4 = _make_segsum(4, with_count=False)


def _tc_layer(h_ch, agg_ch, cnt2, w, b, *, relu, flat_out):
  c_in = h_ch.shape[0]
  d2 = w.shape[0]
  dout = w.shape[1]
  c_out = dout // _L
  rb = 1000
  grid = (_N // rb, c_out)

  def body(h_ref, a_ref, c_ref, w_ref, b_ref, o_ref):
    cnt = c_ref[0, :, :1] + c_ref[1, :, :1]
    inv = 1.0 / jnp.maximum(cnt, 1.0)
    h = jnp.concatenate([h_ref[i] for i in range(c_in)], axis=-1)
    a = jnp.concatenate([a_ref[i] for i in range(c_in)], axis=-1) * inv
    hm = jnp.concatenate([h, a], axis=-1)
    acc = jnp.dot(hm, w_ref[...], preferred_element_type=jnp.float32)
    acc = acc + b_ref[...]
    if relu:
      acc = jnp.maximum(acc, 0.0)
    if flat_out:
      o_ref[...] = acc
    else:
      o_ref[0] = acc

  if flat_out:
    out_shape = jax.ShapeDtypeStruct((_N, dout), jnp.float32)
    out_spec = pl.BlockSpec((rb, _L), lambda i, c: (i, c))
  else:
    out_shape = jax.ShapeDtypeStruct((c_out, _N, _L), jnp.float32)
    out_spec = pl.BlockSpec((1, rb, _L), lambda i, c: (c, i, 0))

  return pl.pallas_call(
      body,
      grid=grid,
      in_specs=[
          pl.BlockSpec((c_in, rb, _L), lambda i, c: (0, i, 0)),
          pl.BlockSpec((c_in, rb, _L), lambda i, c: (0, i, 0)),
          pl.BlockSpec((_NC, rb, _L), lambda i, c: (0, i, 0)),
          pl.BlockSpec((d2, _L), lambda i, c: (0, c)),
          pl.BlockSpec((1, _L), lambda i, c: (0, c)),
      ],
      out_specs=out_spec,
      out_shape=out_shape,
  )(h_ch, agg_ch, cnt2, w, b)


@jax.jit
def kernel(x, edge_index, Ws0, Wn0, b0, Ws1, Wn1, b1, Ws2, Wn2, b2):
  src = edge_index[0].astype(jnp.int32)
  dst = edge_index[1].astype(jnp.int32)
  padn = _EPAD - _E
  srcp = jnp.concatenate([src, jnp.zeros((padn,), jnp.int32)])
  srcp = srcp.reshape(_NS, _NB, _K)
  dstp = jnp.concatenate([dst, jnp.full((padn,), _N, jnp.int32)])
  dstp = dstp.reshape(_NS, _NB, _K)

  x_ch = x.reshape(_N, 2, _L).transpose(1, 0, 2)
  zeros = jnp.zeros((_N, _L), jnp.float32)
  ones = jnp.ones((_K, _L), jnp.float32)

  w0 = jnp.concatenate([Ws0, Wn0], axis=0)
  w1 = jnp.concatenate([Ws1, Wn1], axis=0)
  w2 = jnp.concatenate([Ws2, Wn2], axis=0)

  agg0, cnt2 = _segsum2(x_ch, srcp, dstp, zeros, ones)
  h1 = _tc_layer(x_ch, agg0, cnt2, w0, b0[None], relu=True, flat_out=False)
  agg1, = _segsum4(h1, srcp, dstp, zeros)
  h2 = _tc_layer(h1, agg1, cnt2, w1, b1[None], relu=True, flat_out=False)
  agg2, = _segsum4(h2, srcp, dstp, zeros)
  out = _tc_layer(h2, agg2, cnt2, w2, b2[None], relu=False, flat_out=True)
  return out


# speedup vs baseline: 1.1414x; 1.1414x over previous
import functools

import jax
import jax.numpy as jnp
from jax import lax
from jax.experimental import pallas as pl
from jax.experimental.pallas import tpu as pltpu
from jax.experimental.pallas import tpu_sc as plsc

_N = 10000
_E = 160000
_L = 128
_NC = 2
_NS = 16
_K = 128
_NB = 80
_NH = 5
_NBH = _NB // _NH
_EPT = _NB * _K
_EPAD = _NS * _EPT
_PAD_ROWS = 8
_RPT = 640
_RLAST = _N - (_NS - 1) * _RPT


def _split_rows_copy(s, make_src, make_dst):
  row0 = pl.multiple_of(s * _RPT, 8)

  @pl.when(s < _NS - 1)
  def _():
    pltpu.sync_copy(make_src(row0, _RPT), make_dst(row0, _RPT))

  @pl.when(s == _NS - 1)
  def _():
    off = (_NS - 1) * _RPT
    pltpu.sync_copy(make_src(off, _RLAST), make_dst(off, _RLAST))


def _make_segsum(c_chunks: int, with_count: bool):
  n_pass = c_chunks // _NC
  mesh = plsc.VectorSubcoreMesh(
      core_axis_name="c", subcore_axis_name="s",
      num_cores=_NC, num_subcores=_NS)

  out_type = [jax.ShapeDtypeStruct((c_chunks, _N, _L), jnp.float32)]
  scratch = [
      pltpu.VMEM((_NB, _K), jnp.int32),
      pltpu.VMEM((_NBH, _K), jnp.int32),
      pltpu.VMEM((2, _K, _L), jnp.float32),
      pltpu.VMEM_SHARED((_N + _PAD_ROWS, _L), jnp.float32),
      pltpu.SemaphoreType.DMA,
      pltpu.SemaphoreType.DMA,
      pltpu.SemaphoreType.DMA,
      pltpu.SemaphoreType.DMA,
  ]
  if with_count:
    out_type.append(jax.ShapeDtypeStruct((_NC, _N, _L), jnp.float32))

  def body(*refs):
    if with_count:
      (table, srcp, dstp, zeros, ones,
       out, cnt,
       src_v, dst_v, rows_v, acc, sem0, sem1, sem2, sem3) = refs
    else:
      (table, srcp, dstp, zeros,
       out,
       src_v, dst_v, rows_v, acc, sem0, sem1, sem2, sem3) = refs
    c = lax.axis_index("c")
    s = lax.axis_index("s")

    pltpu.sync_copy(srcp.at[s], src_v)

    for p in range(n_pass):
      chunk = p * _NC + c
      _split_rows_copy(s, lambda o, n: zeros.at[pl.ds(o, n)],
                       lambda o, n: acc.at[pl.ds(o, n)])
      plsc.subcore_barrier()

      def gather(g, buf, sem):
        return pltpu.make_async_copy(
            table.at[chunk].at[src_v.at[g]], rows_v.at[buf], sem)

      def scatter(b, buf, sem):
        return pltpu.make_async_copy(rows_v.at[buf], acc.at[dst_v.at[b]], sem)

      gsem = (sem0, sem1)
      ssem = (sem2, sem3)
      gather(0, 0, sem0).start()
      for half in range(_NH):
        if half > 0:
          scatter(_NBH - 1, (_NBH - 1) % 2, ssem[(_NBH - 1) % 2]).wait()
        pltpu.sync_copy(dstp.at[s, pl.ds(half * _NBH, _NBH)], dst_v)

        def batch_body(b, carry):
          g = half * _NBH + b

          def step(buf):
            other = 1 - buf
            gather(g, buf, gsem[buf]).wait()
            scatter(b, buf, ssem[buf]).start(add=True)

            @pl.when(g + 1 < _NB)
            def _():
              @pl.when(b >= 1)
              def _():
                scatter(b - 1, other, ssem[other]).wait()
              gather(g + 1, other, gsem[other]).start()

          @pl.when(b % 2 == 0)
          def _():
            step(0)

          @pl.when(b % 2 == 1)
          def _():
            step(1)
          return carry
        lax.fori_loop(0, _NBH, batch_body, 0)

      scatter(_NBH - 2, (_NBH - 2) % 2, ssem[(_NBH - 2) % 2]).wait()
      scatter(_NBH - 1, (_NBH - 1) % 2, ssem[(_NBH - 1) % 2]).wait()
      plsc.subcore_barrier()
      _split_rows_copy(s, lambda o, n: acc.at[pl.ds(o, n)],
                       lambda o, n: out.at[chunk, pl.ds(o, n)])
      if p != n_pass - 1:
        plsc.subcore_barrier()

    if with_count:
      plsc.subcore_barrier()
      _split_rows_copy(s, lambda o, n: zeros.at[pl.ds(o, n)],
                       lambda o, n: acc.at[pl.ds(o, n)])
      pltpu.sync_copy(ones, rows_v.at[0])
      plsc.subcore_barrier()
      for half in range(_NH):
        @pl.when(c == half % _NC)
        def _():
          pltpu.sync_copy(dstp.at[s, pl.ds(half * _NBH, _NBH)], dst_v)

          def cnt_start(b, carry):
            pltpu.make_async_copy(
                rows_v.at[0], acc.at[dst_v.at[b]], sem0).start(add=True)
            return carry
          lax.fori_loop(0, _NBH, cnt_start, 0)

          def cnt_drain(b, carry):
            pltpu.make_async_copy(
                rows_v.at[0], acc.at[dst_v.at[b]], sem0).wait()
            return carry
          lax.fori_loop(0, _NBH, cnt_drain, 0)
      plsc.subcore_barrier()
      _split_rows_copy(s, lambda o, n: acc.at[pl.ds(o, n)],
                       lambda o, n: cnt.at[c, pl.ds(o, n)])

  return pl.kernel(body, out_type=out_type, mesh=mesh, scratch_types=scratch)


_segsum2 = _make_segsum(2, with_count=True)
_segsum2nc = _make_segsum(2, with_count=False)
_segsum4 = _make_segsum(4, with_count=False)


def _tc_layer(h_ch, agg_ch, cnt2, w, b, *, relu, flat_out):
  c_in = h_ch.shape[0]
  d2 = w.shape[0]
  dout = w.shape[1]
  c_out = dout // _L
  rb = 1000
  grid = (_N // rb, c_out)

  def body(h_ref, a_ref, c_ref, w_ref, b_ref, o_ref):
    cnt = c_ref[0, :, :1] + c_ref[1, :, :1]
    inv = 1.0 / jnp.maximum(cnt, 1.0)
    h = jnp.concatenate([h_ref[i] for i in range(c_in)], axis=-1)
    a = jnp.concatenate([a_ref[i] for i in range(c_in)], axis=-1) * inv
    hm = jnp.concatenate([h, a], axis=-1)
    acc = jnp.dot(hm, w_ref[...], preferred_element_type=jnp.float32)
    acc = acc + b_ref[...]
    if relu:
      acc = jnp.maximum(acc, 0.0)
    if flat_out:
      o_ref[...] = acc
    else:
      o_ref[0] = acc

  if flat_out:
    out_shape = jax.ShapeDtypeStruct((_N, dout), jnp.float32)
    out_spec = pl.BlockSpec((rb, _L), lambda i, c: (i, c))
  else:
    out_shape = jax.ShapeDtypeStruct((c_out, _N, _L), jnp.float32)
    out_spec = pl.BlockSpec((1, rb, _L), lambda i, c: (c, i, 0))

  return pl.pallas_call(
      body,
      grid=grid,
      in_specs=[
          pl.BlockSpec((c_in, rb, _L), lambda i, c: (0, i, 0)),
          pl.BlockSpec((c_in, rb, _L), lambda i, c: (0, i, 0)),
          pl.BlockSpec((_NC, rb, _L), lambda i, c: (0, i, 0)),
          pl.BlockSpec((d2, _L), lambda i, c: (0, c)),
          pl.BlockSpec((1, _L), lambda i, c: (0, c)),
      ],
      out_specs=out_spec,
      out_shape=out_shape,
  )(h_ch, agg_ch, cnt2, w, b)


def _tc_layer1_prep(h_ch, agg_ch, cnt2, w, b, ws2, wn2):
  c_in = h_ch.shape[0]
  dout = w.shape[1]
  c_out = dout // _L
  rb = 1000
  grid = (_N // rb, c_out)

  def body(h_ref, a_ref, c_ref, w_ref, b_ref, ws2_ref, wn2_ref,
           z_ref, s_ref):
    cnt = c_ref[0, :, :1] + c_ref[1, :, :1]
    inv = 1.0 / jnp.maximum(cnt, 1.0)
    h = jnp.concatenate([h_ref[i] for i in range(c_in)], axis=-1)
    a = jnp.concatenate([a_ref[i] for i in range(c_in)], axis=-1) * inv
    hm = jnp.concatenate([h, a], axis=-1)
    h2 = jnp.dot(hm, w_ref[...], preferred_element_type=jnp.float32)
    h2 = jnp.maximum(h2 + b_ref[...], 0.0)
    z = jnp.dot(h2, wn2_ref[...], preferred_element_type=jnp.float32)
    sf = jnp.dot(h2, ws2_ref[...], preferred_element_type=jnp.float32)
    cc = pl.program_id(1)

    @pl.when(cc == 0)
    def _():
      z_ref[0] = z[:, :_L]
      z_ref[1] = z[:, _L:]
      s_ref[...] = sf

    @pl.when(cc != 0)
    def _():
      z_ref[0] += z[:, :_L]
      z_ref[1] += z[:, _L:]
      s_ref[...] += sf

  d2 = w.shape[0]
  dl = ws2.shape[1]
  return pl.pallas_call(
      body,
      grid=grid,
      in_specs=[
          pl.BlockSpec((c_in, rb, _L), lambda i, c: (0, i, 0)),
          pl.BlockSpec((c_in, rb, _L), lambda i, c: (0, i, 0)),
          pl.BlockSpec((_NC, rb, _L), lambda i, c: (0, i, 0)),
          pl.BlockSpec((d2, _L), lambda i, c: (0, c)),
          pl.BlockSpec((1, _L), lambda i, c: (0, c)),
          pl.BlockSpec((_L, dl), lambda i, c: (c, 0)),
          pl.BlockSpec((_L, dl), lambda i, c: (c, 0)),
      ],
      out_specs=[
          pl.BlockSpec((2, rb, _L), lambda i, c: (0, i, 0)),
          pl.BlockSpec((rb, dl), lambda i, c: (i, 0)),
      ],
      out_shape=[
          jax.ShapeDtypeStruct((2, _N, _L), jnp.float32),
          jax.ShapeDtypeStruct((_N, dl), jnp.float32),
      ],
  )(h_ch, agg_ch, cnt2, w, b, ws2, wn2)


def _tc_combine(self2, agg_ch, cnt2, b):
  rb = 1000
  dl = self2.shape[1]
  grid = (_N // rb,)

  def body(s_ref, a_ref, c_ref, b_ref, o_ref):
    cnt = c_ref[0, :, :1] + c_ref[1, :, :1]
    inv = 1.0 / jnp.maximum(cnt, 1.0)
    mean = jnp.concatenate([a_ref[0], a_ref[1]], axis=-1) * inv
    o_ref[...] = s_ref[...] + mean + b_ref[...]

  return pl.pallas_call(
      body,
      grid=grid,
      in_specs=[
          pl.BlockSpec((rb, dl), lambda i: (i, 0)),
          pl.BlockSpec((2, rb, _L), lambda i: (0, i, 0)),
          pl.BlockSpec((_NC, rb, _L), lambda i: (0, i, 0)),
          pl.BlockSpec((1, dl), lambda i: (0, 0)),
      ],
      out_specs=pl.BlockSpec((rb, dl), lambda i: (i, 0)),
      out_shape=jax.ShapeDtypeStruct((_N, dl), jnp.float32),
  )(self2, agg_ch, cnt2, b)


@jax.jit
def kernel(x, edge_index, Ws0, Wn0, b0, Ws1, Wn1, b1, Ws2, Wn2, b2):
  src = edge_index[0].astype(jnp.int32)
  dst = edge_index[1].astype(jnp.int32)
  padn = _EPAD - _E
  srcp = jnp.concatenate([src, jnp.zeros((padn,), jnp.int32)])
  srcp = srcp.reshape(_NS, _NB, _K)
  dstp = jnp.concatenate([dst, jnp.full((padn,), _N, jnp.int32)])
  dstp = dstp.reshape(_NS, _NB, _K)

  x_ch = x.reshape(_N, 2, _L).transpose(1, 0, 2)
  zeros = jnp.zeros((_N, _L), jnp.float32)
  ones = jnp.ones((_K, _L), jnp.float32)

  w0 = jnp.concatenate([Ws0, Wn0], axis=0)
  w1 = jnp.concatenate([Ws1, Wn1], axis=0)

  agg0, cnt2 = _segsum2(x_ch, srcp, dstp, zeros, ones)
  h1 = _tc_layer(x_ch, agg0, cnt2, w0, b0[None], relu=True, flat_out=False)
  agg1, = _segsum4(h1, srcp, dstp, zeros)
  z2, self2 = _tc_layer1_prep(h1, agg1, cnt2, w1, b1[None], Ws2, Wn2)
  agg2, = _segsum2nc(z2, srcp, dstp, zeros)
  out = _tc_combine(self2, agg2, cnt2, b2[None])
  return out


# speedup vs baseline: 1.1767x; 1.0310x over previous
import functools

import jax
import jax.numpy as jnp
from jax import lax
from jax.experimental import pallas as pl
from jax.experimental.pallas import tpu as pltpu
from jax.experimental.pallas import tpu_sc as plsc

_N = 10000
_E = 160000
_L = 128
_NC = 2
_NS = 16
_K = 128
_NB = 80
_NH = 5
_NBH = _NB // _NH
_EPT = _NB * _K
_EPAD = _NS * _EPT
_PAD_ROWS = 8
_RPT = 640
_RLAST = _N - (_NS - 1) * _RPT


def _split_rows_copy(s, make_src, make_dst):
  row0 = pl.multiple_of(s * _RPT, 8)

  @pl.when(s < _NS - 1)
  def _():
    pltpu.sync_copy(make_src(row0, _RPT), make_dst(row0, _RPT))

  @pl.when(s == _NS - 1)
  def _():
    off = (_NS - 1) * _RPT
    pltpu.sync_copy(make_src(off, _RLAST), make_dst(off, _RLAST))


def _make_segsum(c_chunks: int, with_count: bool):
  n_pass = c_chunks // _NC
  mesh = plsc.VectorSubcoreMesh(
      core_axis_name="c", subcore_axis_name="s",
      num_cores=_NC, num_subcores=_NS)

  out_type = [jax.ShapeDtypeStruct((c_chunks, _N, _L), jnp.float32)]
  scratch = [
      pltpu.VMEM((_NB, _K), jnp.int32),
      pltpu.VMEM((_NBH, _K), jnp.int32),
      pltpu.VMEM((2, _K, _L), jnp.float32),
      pltpu.VMEM_SHARED((_N + _PAD_ROWS, _L), jnp.float32),
      pltpu.SemaphoreType.DMA,
      pltpu.SemaphoreType.DMA,
      pltpu.SemaphoreType.DMA,
      pltpu.SemaphoreType.DMA,
  ]
  if with_count:
    out_type.append(jax.ShapeDtypeStruct((_NC, _N, _L), jnp.float32))

  def body(*refs):
    if with_count:
      (table, srcp, dstp, zeros, ones,
       out, cnt,
       src_v, dst_v, rows_v, acc, sem0, sem1, sem2, sem3) = refs
    else:
      (table, srcp, dstp, zeros,
       out,
       src_v, dst_v, rows_v, acc, sem0, sem1, sem2, sem3) = refs
    c = lax.axis_index("c")
    s = lax.axis_index("s")

    pltpu.sync_copy(srcp.at[s], src_v)

    for p in range(n_pass):
      chunk = p * _NC + c
      _split_rows_copy(s, lambda o, n: zeros.at[pl.ds(o, n)],
                       lambda o, n: acc.at[pl.ds(o, n)])
      plsc.subcore_barrier()

      def gather(g, buf, sem):
        return pltpu.make_async_copy(
            table.at[chunk].at[src_v.at[g]], rows_v.at[buf], sem)

      def scatter(b, buf, sem):
        return pltpu.make_async_copy(rows_v.at[buf], acc.at[dst_v.at[b]], sem)

      gsem = (sem0, sem1)
      ssem = (sem2, sem3)
      gather(0, 0, sem0).start()
      for half in range(_NH):
        if half > 0:
          scatter(_NBH - 1, (_NBH - 1) % 2, ssem[(_NBH - 1) % 2]).wait()
        pltpu.sync_copy(dstp.at[s, pl.ds(half * _NBH, _NBH)], dst_v)

        def batch_body(b, carry):
          g = half * _NBH + b

          def step(buf):
            other = 1 - buf
            gather(g, buf, gsem[buf]).wait()
            scatter(b, buf, ssem[buf]).start(add=True)

            @pl.when(g + 1 < _NB)
            def _():
              @pl.when(b >= 1)
              def _():
                scatter(b - 1, other, ssem[other]).wait()
              gather(g + 1, other, gsem[other]).start()

          @pl.when(b % 2 == 0)
          def _():
            step(0)

          @pl.when(b % 2 == 1)
          def _():
            step(1)
          return carry
        lax.fori_loop(0, _NBH, batch_body, 0)

      scatter(_NBH - 2, (_NBH - 2) % 2, ssem[(_NBH - 2) % 2]).wait()
      scatter(_NBH - 1, (_NBH - 1) % 2, ssem[(_NBH - 1) % 2]).wait()
      plsc.subcore_barrier()
      _split_rows_copy(s, lambda o, n: acc.at[pl.ds(o, n)],
                       lambda o, n: out.at[chunk, pl.ds(o, n)])
      if p != n_pass - 1:
        plsc.subcore_barrier()

    if with_count:
      plsc.subcore_barrier()
      _split_rows_copy(s, lambda o, n: zeros.at[pl.ds(o, n)],
                       lambda o, n: acc.at[pl.ds(o, n)])
      pltpu.sync_copy(ones, rows_v.at[0])
      plsc.subcore_barrier()
      for half in range(_NH):
        @pl.when(c == half % _NC)
        def _():
          pltpu.sync_copy(dstp.at[s, pl.ds(half * _NBH, _NBH)], dst_v)

          def cnt_start(b, carry):
            pltpu.make_async_copy(
                rows_v.at[0], acc.at[dst_v.at[b]], sem0).start(add=True)
            return carry
          lax.fori_loop(0, _NBH, cnt_start, 0)

          def cnt_drain(b, carry):
            pltpu.make_async_copy(
                rows_v.at[0], acc.at[dst_v.at[b]], sem0).wait()
            return carry
          lax.fori_loop(0, _NBH, cnt_drain, 0)
      plsc.subcore_barrier()
      _split_rows_copy(s, lambda o, n: acc.at[pl.ds(o, n)],
                       lambda o, n: cnt.at[c, pl.ds(o, n)])

  return pl.kernel(body, out_type=out_type, mesh=mesh, scratch_types=scratch)


_segsum2 = _make_segsum(2, with_count=True)
_segsum2nc = _make_segsum(2, with_count=False)
_segsum4 = _make_segsum(4, with_count=False)


def _tc_layer(h_ch, agg_ch, cnt2, w, b, *, relu, flat_out):
  c_in = h_ch.shape[0]
  d2 = w.shape[0]
  dout = w.shape[1]
  wl = 2 * _L
  rb = 1000
  grid = (_N // rb, dout // wl)

  def body(h_ref, a_ref, c_ref, w_ref, b_ref, o_ref):
    cnt = c_ref[0, :, :1] + c_ref[1, :, :1]
    inv = 1.0 / jnp.maximum(cnt, 1.0)
    h = jnp.concatenate([h_ref[i] for i in range(c_in)], axis=-1)
    a = jnp.concatenate([a_ref[i] for i in range(c_in)], axis=-1) * inv
    hm = jnp.concatenate([h, a], axis=-1)
    acc = jnp.dot(hm, w_ref[...], preferred_element_type=jnp.float32)
    acc = acc + b_ref[...]
    if relu:
      acc = jnp.maximum(acc, 0.0)
    if flat_out:
      o_ref[...] = acc
    else:
      o_ref[0] = acc[:, :_L]
      o_ref[1] = acc[:, _L:]

  if flat_out:
    out_shape = jax.ShapeDtypeStruct((_N, dout), jnp.float32)
    out_spec = pl.BlockSpec((rb, wl), lambda i, c: (i, c))
  else:
    out_shape = jax.ShapeDtypeStruct((dout // _L, _N, _L), jnp.float32)
    out_spec = pl.BlockSpec((2, rb, _L), lambda i, c: (c, i, 0))

  return pl.pallas_call(
      body,
      grid=grid,
      in_specs=[
          pl.BlockSpec((c_in, rb, _L), lambda i, c: (0, i, 0)),
          pl.BlockSpec((c_in, rb, _L), lambda i, c: (0, i, 0)),
          pl.BlockSpec((_NC, rb, _L), lambda i, c: (0, i, 0)),
          pl.BlockSpec((d2, wl), lambda i, c: (0, c)),
          pl.BlockSpec((1, wl), lambda i, c: (0, c)),
      ],
      out_specs=out_spec,
      out_shape=out_shape,
  )(h_ch, agg_ch, cnt2, w, b)


def _tc_layer1_prep(h_ch, agg_ch, cnt2, w, b, ws2, wn2):
  c_in = h_ch.shape[0]
  dout = w.shape[1]
  wl = 2 * _L
  rb = 1000
  grid = (_N // rb, dout // wl)

  def body(h_ref, a_ref, c_ref, w_ref, b_ref, ws2_ref, wn2_ref,
           z_ref, s_ref):
    cnt = c_ref[0, :, :1] + c_ref[1, :, :1]
    inv = 1.0 / jnp.maximum(cnt, 1.0)
    h = jnp.concatenate([h_ref[i] for i in range(c_in)], axis=-1)
    a = jnp.concatenate([a_ref[i] for i in range(c_in)], axis=-1) * inv
    hm = jnp.concatenate([h, a], axis=-1)
    h2 = jnp.dot(hm, w_ref[...], preferred_element_type=jnp.float32)
    h2 = jnp.maximum(h2 + b_ref[...], 0.0)
    z = jnp.dot(h2, wn2_ref[...], preferred_element_type=jnp.float32)
    sf = jnp.dot(h2, ws2_ref[...], preferred_element_type=jnp.float32)
    cc = pl.program_id(1)

    @pl.when(cc == 0)
    def _():
      z_ref[0] = z[:, :_L]
      z_ref[1] = z[:, _L:]
      s_ref[...] = sf

    @pl.when(cc != 0)
    def _():
      z_ref[0] += z[:, :_L]
      z_ref[1] += z[:, _L:]
      s_ref[...] += sf

  d2 = w.shape[0]
  dl = ws2.shape[1]
  return pl.pallas_call(
      body,
      grid=grid,
      in_specs=[
          pl.BlockSpec((c_in, rb, _L), lambda i, c: (0, i, 0)),
          pl.BlockSpec((c_in, rb, _L), lambda i, c: (0, i, 0)),
          pl.BlockSpec((_NC, rb, _L), lambda i, c: (0, i, 0)),
          pl.BlockSpec((d2, wl), lambda i, c: (0, c)),
          pl.BlockSpec((1, wl), lambda i, c: (0, c)),
          pl.BlockSpec((wl, dl), lambda i, c: (c, 0)),
          pl.BlockSpec((wl, dl), lambda i, c: (c, 0)),
      ],
      out_specs=[
          pl.BlockSpec((2, rb, _L), lambda i, c: (0, i, 0)),
          pl.BlockSpec((rb, dl), lambda i, c: (i, 0)),
      ],
      out_shape=[
          jax.ShapeDtypeStruct((2, _N, _L), jnp.float32),
          jax.ShapeDtypeStruct((_N, dl), jnp.float32),
      ],
  )(h_ch, agg_ch, cnt2, w, b, ws2, wn2)


def _tc_combine(self2, agg_ch, cnt2, b):
  rb = 1000
  dl = self2.shape[1]
  grid = (_N // rb,)

  def body(s_ref, a_ref, c_ref, b_ref, o_ref):
    cnt = c_ref[0, :, :1] + c_ref[1, :, :1]
    inv = 1.0 / jnp.maximum(cnt, 1.0)
    mean = jnp.concatenate([a_ref[0], a_ref[1]], axis=-1) * inv
    o_ref[...] = s_ref[...] + mean + b_ref[...]

  return pl.pallas_call(
      body,
      grid=grid,
      in_specs=[
          pl.BlockSpec((rb, dl), lambda i: (i, 0)),
          pl.BlockSpec((2, rb, _L), lambda i: (0, i, 0)),
          pl.BlockSpec((_NC, rb, _L), lambda i: (0, i, 0)),
          pl.BlockSpec((1, dl), lambda i: (0, 0)),
      ],
      out_specs=pl.BlockSpec((rb, dl), lambda i: (i, 0)),
      out_shape=jax.ShapeDtypeStruct((_N, dl), jnp.float32),
  )(self2, agg_ch, cnt2, b)


@jax.jit
def kernel(x, edge_index, Ws0, Wn0, b0, Ws1, Wn1, b1, Ws2, Wn2, b2):
  src = edge_index[0].astype(jnp.int32)
  dst = edge_index[1].astype(jnp.int32)
  padn = _EPAD - _E
  srcp = jnp.concatenate([src, jnp.zeros((padn,), jnp.int32)])
  srcp = srcp.reshape(_NS, _NB, _K)
  dstp = jnp.concatenate([dst, jnp.full((padn,), _N, jnp.int32)])
  dstp = dstp.reshape(_NS, _NB, _K)

  x_ch = x.reshape(_N, 2, _L).transpose(1, 0, 2)
  zeros = jnp.zeros((_N, _L), jnp.float32)
  ones = jnp.ones((_K, _L), jnp.float32)

  w0 = jnp.concatenate([Ws0, Wn0], axis=0)
  w1 = jnp.concatenate([Ws1, Wn1], axis=0)

  agg0, cnt2 = _segsum2(x_ch, srcp, dstp, zeros, ones)
  h1 = _tc_layer(x_ch, agg0, cnt2, w0, b0[None], relu=True, flat_out=False)
  agg1, = _segsum4(h1, srcp, dstp, zeros)
  z2, self2 = _tc_layer1_prep(h1, agg1, cnt2, w1, b1[None], Ws2, Wn2)
  agg2, = _segsum2nc(z2, srcp, dstp, zeros)
  out = _tc_combine(self2, agg2, cnt2, b2[None])
  return out
